# baseline (speedup 1.0000x reference)
import functools

import jax
import jax.numpy as jnp
from jax import lax
from jax.experimental import pallas as pl
from jax.experimental.pallas import tpu as pltpu
from jax.experimental.pallas import tpu_sc as plsc

DIM = 32
LANES = 16
NUM_WORKERS = 32
IDX_ROW = 128
CHUNK_IROWS = 8
CHUNK = IDX_ROW * CHUNK_IROWS
GROUPS = CHUNK // LANES
EPS = 1e-12


def _rsqrt(x):
    bits = lax.bitcast_convert_type(x, jnp.int32)
    y = lax.bitcast_convert_type(0x5F3759DF - (bits >> 1), jnp.float32)
    for _ in range(3):
        y = y * (1.5 - 0.5 * x * y * y)
    return y


def _sc_body(seq_hbm, table_hbm, gamma_hbm, beta_hbm, out_hbm,
             idx_v, rows_v, gb_v, gbc_v, sem):
    cid = lax.axis_index("c")
    sid = lax.axis_index("s")
    wid = sid * 2 + cid

    iota = lax.iota(jnp.int32, LANES)

    pltpu.sync_copy(gamma_hbm, gb_v.at[0])
    pltpu.sync_copy(beta_hbm, gb_v.at[1])
    for d in range(DIM):
        col = jnp.full((LANES,), d, jnp.int32)
        gbc_v[0, d] = plsc.load_gather(gb_v, [jnp.zeros((LANES,), jnp.int32),
                                              col])
        gbc_v[1, d] = plsc.load_gather(gb_v, [jnp.ones((LANES,), jnp.int32),
                                              col])

    total_irows = seq_hbm.shape[0]
    irows_per_worker = total_irows // NUM_WORKERS
    n_chunks = irows_per_worker // CHUNK_IROWS
    irow_base = wid * irows_per_worker

    def chunk_body(ch, carry):
        irow0 = irow_base + ch * CHUNK_IROWS
        pltpu.sync_copy(seq_hbm.at[pl.ds(irow0, CHUNK_IROWS)], idx_v)
        cps = []
        for j in range(CHUNK_IROWS):
            cps.append(pltpu.async_copy(
                table_hbm.at[idx_v.at[j]],
                rows_v.at[pl.ds(j * IDX_ROW, IDX_ROW)],
                sem))
        for cp in cps:
            cp.wait()

        def group_body(g, gc):
            row_idx = g * LANES + iota
            s = jnp.zeros((LANES,), jnp.float32)
            q = jnp.zeros((LANES,), jnp.float32)
            xs = []
            for d in range(DIM):
                col = jnp.full((LANES,), d, jnp.int32)
                xd = plsc.load_gather(rows_v, [row_idx, col])
                xs.append(xd)
                s = s + xd
                q = q + xd * xd
            mean = s * (1.0 / DIM)
            var = q * (1.0 / DIM) - mean * mean
            a = _rsqrt(jnp.maximum(var, 0.0) + EPS)
            ma = mean * a
            for d in range(DIM):
                y = (xs[d] * a - ma) * gbc_v[0, d] + gbc_v[1, d]
                y = jnp.where(y > 0.0, y, jnp.exp(y) - 1.0)
                plsc.store_scatter(rows_v,
                                   [row_idx, jnp.full((LANES,), d, jnp.int32)],
                                   y)
            return gc

        lax.fori_loop(0, GROUPS, group_body, 0)
        pltpu.sync_copy(rows_v, out_hbm.at[pl.ds(irow0 * IDX_ROW, CHUNK)])
        return carry

    lax.fori_loop(0, n_chunks, chunk_body, 0)


def _make_sc_call(n_rows):
    return functools.partial(
        pl.kernel,
        out_type=jax.ShapeDtypeStruct((n_rows, DIM), jnp.float32),
        mesh=plsc.VectorSubcoreMesh(core_axis_name="c", subcore_axis_name="s"),
        compiler_params=pltpu.CompilerParams(needs_layout_passes=False,
                                             use_tc_tiling_on_sc=False),
        scratch_types=[
            pltpu.VMEM((CHUNK_IROWS, IDX_ROW), jnp.int32),
            pltpu.VMEM((CHUNK, DIM), jnp.float32),
            pltpu.VMEM((2, DIM), jnp.float32),
            pltpu.VMEM((2, DIM, LANES), jnp.float32),
            pltpu.SemaphoreType.DMA,
        ],
    )(_sc_body)


@jax.jit
def kernel(seq, table, gamma, beta):
    bsz, seqlen = seq.shape
    n_rows = bsz * seqlen
    seq2d = seq.reshape(n_rows // IDX_ROW, IDX_ROW).astype(jnp.int32)
    out = _make_sc_call(n_rows)(seq2d, table, gamma, beta)
    return out.reshape(bsz, seqlen, DIM)


# speedup vs baseline: 1.7962x; 1.7962x over previous
import functools

import jax
import jax.numpy as jnp
from jax import lax
from jax.experimental import pallas as pl
from jax.experimental.pallas import tpu as pltpu
from jax.experimental.pallas import tpu_sc as plsc

DIM = 32
LANES = 16
NUM_WORKERS = 32
IDX_ROW = 128
CHUNK_IROWS = 4
CHUNK = IDX_ROW * CHUNK_IROWS
GROUPS = CHUNK // LANES
EPS = 1e-12


def _rsqrt(x):
    bits = lax.bitcast_convert_type(x, jnp.int32)
    y = lax.bitcast_convert_type(0x5F3759DF - (bits >> 1), jnp.float32)
    for _ in range(2):
        y = y * (1.5 - 0.5 * x * y * y)
    return y


def _splat(vec, lane):
    idx = jnp.full((LANES, 1), lane, jnp.int32)
    return lax.gather(
        vec, idx,
        dimension_numbers=lax.GatherDimensionNumbers(
            offset_dims=(), collapsed_slice_dims=(0,), start_index_map=(0,)),
        slice_sizes=(1,),
        mode=lax.GatherScatterMode.PROMISE_IN_BOUNDS)


def _sc_body(seq_hbm, table_hbm, gamma_hbm, beta_hbm, out_hbm,
             idx_v, rows_v, out_v, gb_v, gsem0, gsem1, osem0, osem1):
    cid = lax.axis_index("c")
    sid = lax.axis_index("s")
    wid = sid * 2 + cid
    gsems = (gsem0, gsem1)
    osems = (osem0, osem1)

    iota = lax.iota(jnp.int32, LANES)

    pltpu.sync_copy(gamma_hbm, gb_v.at[0])
    pltpu.sync_copy(beta_hbm, gb_v.at[1])
    g0 = gb_v[0, pl.ds(0, LANES)]
    g1 = gb_v[0, pl.ds(LANES, LANES)]
    b0 = gb_v[1, pl.ds(0, LANES)]
    b1 = gb_v[1, pl.ds(LANES, LANES)]

    total_irows = seq_hbm.shape[0]
    irows_per_worker = total_irows // NUM_WORKERS
    n_chunks = irows_per_worker // CHUNK_IROWS
    irow_base = wid * irows_per_worker

    def fire_chunk(ch, b):
        irow0 = irow_base + ch * CHUNK_IROWS
        pltpu.sync_copy(seq_hbm.at[pl.ds(irow0, CHUNK_IROWS)], idx_v.at[b])
        for j in range(CHUNK_IROWS):
            pltpu.async_copy(
                table_hbm.at[idx_v.at[b, j]],
                rows_v.at[b, pl.ds(j * IDX_ROW, IDX_ROW)],
                gsems[b])

    def wait_gathers(b):
        pltpu.make_async_copy(table_hbm.at[pl.ds(0, CHUNK)],
                              rows_v.at[b], gsems[b]).wait()

    def fire_out(ch, b):
        irow0 = irow_base + ch * CHUNK_IROWS
        pltpu.async_copy(out_v.at[b],
                         out_hbm.at[pl.ds(irow0 * IDX_ROW, CHUNK)],
                         osems[b])

    def wait_out(b):
        pltpu.make_async_copy(out_v.at[b],
                              out_hbm.at[pl.ds(0, CHUNK)], osems[b]).wait()

    def compute(b):
        def group_body(g):
            row0 = g * LANES
            row_idx = row0 + iota
            s = jnp.zeros((LANES,), jnp.float32)
            q = jnp.zeros((LANES,), jnp.float32)
            for d in range(DIM):
                xd = plsc.load_gather(rows_v.at[b],
                                      [row_idx,
                                       jnp.full((LANES,), d, jnp.int32)])
                s = s + xd
                q = q + xd * xd
            mean = s * (1.0 / DIM)
            var = q * (1.0 / DIM) - mean * mean
            a = _rsqrt(jnp.maximum(var, 0.0) + EPS)
            ma = mean * a
            for r0 in range(0, LANES, 8):
                rows = [row0 + r0 + k for k in range(8)]
                vs = []
                for k in range(8):
                    vs.append(rows_v[b, rows[k], pl.ds(0, LANES)])
                    vs.append(rows_v[b, rows[k], pl.ds(LANES, LANES)])
                ys = []
                for k in range(8):
                    ar = _splat(a, r0 + k)
                    mar = _splat(ma, r0 + k)
                    ys.append((vs[2 * k] * ar - mar) * g0 + b0)
                    ys.append((vs[2 * k + 1] * ar - mar) * g1 + b1)
                es = [jnp.exp(y) for y in ys]
                ys = [jnp.where(y > 0.0, y, e - 1.0)
                      for y, e in zip(ys, es)]
                for k in range(8):
                    out_v[b, rows[k], pl.ds(0, LANES)] = ys[2 * k]
                    out_v[b, rows[k], pl.ds(LANES, LANES)] = ys[2 * k + 1]

        plsc.parallel_loop(0, GROUPS, 1, unroll=2)(group_body)

    fire_chunk(0, 0)
    fire_chunk(1, 1)

    def outer(i, carry):
        for b in range(2):
            ch = 2 * i + b

            @pl.when(i >= 1)
            def _wait_prev_out():
                wait_out(b)

            wait_gathers(b)
            compute(b)
            fire_out(ch, b)

            @pl.when(i < (n_chunks // 2) - 1)
            def _prefetch():
                fire_chunk(ch + 2, b)
        return carry

    lax.fori_loop(0, n_chunks // 2, outer, 0)
    wait_out(0)
    wait_out(1)


def _make_sc_call(n_rows):
    return functools.partial(
        pl.kernel,
        out_type=jax.ShapeDtypeStruct((n_rows, DIM), jnp.float32),
        mesh=plsc.VectorSubcoreMesh(core_axis_name="c", subcore_axis_name="s"),
        compiler_params=pltpu.CompilerParams(needs_layout_passes=False,
                                             use_tc_tiling_on_sc=False),
        scratch_types=[
            pltpu.VMEM((2, CHUNK_IROWS, IDX_ROW), jnp.int32),
            pltpu.VMEM((2, CHUNK, DIM), jnp.float32),
            pltpu.VMEM((2, CHUNK, DIM), jnp.float32),
            pltpu.VMEM((2, DIM), jnp.float32),
            pltpu.SemaphoreType.DMA,
            pltpu.SemaphoreType.DMA,
            pltpu.SemaphoreType.DMA,
            pltpu.SemaphoreType.DMA,
        ],
    )(_sc_body)


@jax.jit
def kernel(seq, table, gamma, beta):
    bsz, seqlen = seq.shape
    n_rows = bsz * seqlen
    seq2d = seq.reshape(n_rows // IDX_ROW, IDX_ROW).astype(jnp.int32)
    out = _make_sc_call(n_rows)(seq2d, table, gamma, beta)
    return out.reshape(bsz, seqlen, DIM)
